# Initial kernel scaffold; baseline (speedup 1.0000x reference)
import jax
import jax.numpy as jnp
from jax.experimental import pallas as pl


def kernel(input_ids, word_embeddings, position_embeddings, proj_weight, ln_gamma, ln_beta):
    raise NotImplementedError("write your pallas kernel here")


# speedup vs baseline: 1.8616x; 1.8616x over previous
import functools

import jax
import jax.numpy as jnp
from jax import lax
from jax.experimental import pallas as pl
from jax.experimental.pallas import tpu as pltpu
from jax.experimental.pallas import tpu_sc as plsc

VOCAB = 128100
EMB = 512
HID = 1024
B = 4
S = 2048
EPS = 1e-07

N_TOK = B * S

_CHUNK = 128


def _make_sc_gather():
    info = plsc.get_sparse_core_info()
    nc, ns = info.num_cores, info.num_subcores
    nw = nc * ns
    per_w = N_TOK // nw
    n_chunks = per_w // _CHUNK
    mesh = plsc.VectorSubcoreMesh(core_axis_name="c", subcore_axis_name="s")

    @functools.partial(
        pl.kernel,
        mesh=mesh,
        out_type=jax.ShapeDtypeStruct((N_TOK, EMB), jnp.float32),
        scratch_types=[
            pltpu.VMEM((_CHUNK,), jnp.int32),
            pltpu.VMEM((_CHUNK, EMB), jnp.float32),
            pltpu.SemaphoreType.DMA,
        ],
    )
    def gather_k(idx_hbm, table_hbm, out_hbm, idx_v, rows_v, sem):
        wid = lax.axis_index("s") * nc + lax.axis_index("c")
        base0 = wid * per_w
        for c in range(n_chunks):
            base = base0 + c * _CHUNK
            pltpu.sync_copy(idx_hbm.at[pl.ds(base, _CHUNK)], idx_v)
            pltpu.async_copy(table_hbm.at[idx_v], rows_v, sem).wait()
            pltpu.sync_copy(rows_v, out_hbm.at[pl.ds(base, _CHUNK)])

    return gather_k


_BLK = 512


def _tc_body(g_ref, p_ref, w_ref, gamma_ref, beta_ref, o_ref):
    x = g_ref[...] + p_ref[...]
    y = lax.dot_general(x, w_ref[...], (((1,), (1,)), ((), ())),
                        preferred_element_type=jnp.float32)
    mean = jnp.mean(y, axis=-1, keepdims=True)
    yc = y - mean
    var = jnp.mean(yc * yc, axis=-1, keepdims=True)
    o_ref[...] = yc * lax.rsqrt(var + EPS) * gamma_ref[...] + beta_ref[...]


def _tc_call(gathered, pos, w, gamma, beta):
    grid = (N_TOK // _BLK,)
    s_blocks = S // _BLK
    return pl.pallas_call(
        _tc_body,
        grid=grid,
        in_specs=[
            pl.BlockSpec((_BLK, EMB), lambda i: (i, 0)),
            pl.BlockSpec((_BLK, EMB), lambda i: (i % s_blocks, 0)),
            pl.BlockSpec((HID, EMB), lambda i: (0, 0)),
            pl.BlockSpec((1, HID), lambda i: (0, 0)),
            pl.BlockSpec((1, HID), lambda i: (0, 0)),
        ],
        out_specs=pl.BlockSpec((_BLK, HID), lambda i: (i, 0)),
        out_shape=jax.ShapeDtypeStruct((N_TOK, HID), jnp.float32),
    )(gathered, pos, w, gamma, beta)


def kernel(input_ids, word_embeddings, position_embeddings, proj_weight, ln_gamma, ln_beta):
    ids_flat = input_ids.reshape(N_TOK)
    gathered = _make_sc_gather()(ids_flat, word_embeddings)
    out = _tc_call(
        gathered,
        position_embeddings,
        proj_weight,
        ln_gamma.reshape(1, HID),
        ln_beta.reshape(1, HID),
    )
    return out.reshape(B, S, HID)


# speedup vs baseline: 1.9253x; 1.0342x over previous
import functools

import jax
import jax.numpy as jnp
from jax import lax
from jax.experimental import pallas as pl
from jax.experimental.pallas import tpu as pltpu
from jax.experimental.pallas import tpu_sc as plsc

VOCAB = 128100
EMB = 512
HID = 1024
B = 4
S = 2048
EPS = 1e-07

N_TOK = B * S

_CHUNK = 128


def _make_sc_gather():
    info = plsc.get_sparse_core_info()
    nc, ns = info.num_cores, info.num_subcores
    nw = nc * ns
    per_w = N_TOK // nw
    n_chunks = per_w // _CHUNK
    mesh = plsc.VectorSubcoreMesh(core_axis_name="c", subcore_axis_name="s")

    @functools.partial(
        pl.kernel,
        mesh=mesh,
        out_type=jax.ShapeDtypeStruct((N_TOK, EMB), jnp.float32),
        scratch_types=[
            pltpu.VMEM((_CHUNK,), jnp.int32),
            pltpu.VMEM((_CHUNK, EMB), jnp.float32),
            pltpu.SemaphoreType.DMA,
        ],
    )
    def gather_k(idx_hbm, table_hbm, out_hbm, idx_v, rows_v, sem):
        wid = lax.axis_index("s") * nc + lax.axis_index("c")
        base0 = wid * per_w
        for c in range(n_chunks):
            base = base0 + c * _CHUNK
            pltpu.sync_copy(idx_hbm.at[pl.ds(base, _CHUNK)], idx_v)
            pltpu.async_copy(table_hbm.at[idx_v], rows_v, sem).wait()
            pltpu.sync_copy(rows_v, out_hbm.at[pl.ds(base, _CHUNK)])

    return gather_k


_BLK = 512


def _tc_body(g_ref, p_ref, w_ref, gamma_ref, beta_ref, o_ref):
    x = (g_ref[...] + p_ref[...]).astype(jnp.bfloat16)
    y = lax.dot_general(x, w_ref[...].astype(jnp.bfloat16),
                        (((1,), (1,)), ((), ())),
                        preferred_element_type=jnp.float32)
    mean = jnp.mean(y, axis=-1, keepdims=True)
    yc = y - mean
    var = jnp.mean(yc * yc, axis=-1, keepdims=True)
    o_ref[...] = yc * lax.rsqrt(var + EPS) * gamma_ref[...] + beta_ref[...]


def _tc_call(gathered, pos, w, gamma, beta):
    s_blocks = S // _BLK
    return pl.pallas_call(
        _tc_body,
        grid=(s_blocks, B),
        in_specs=[
            pl.BlockSpec((_BLK, EMB), lambda i, j: (j * s_blocks + i, 0)),
            pl.BlockSpec((_BLK, EMB), lambda i, j: (i, 0)),
            pl.BlockSpec((HID, EMB), lambda i, j: (0, 0)),
            pl.BlockSpec((1, HID), lambda i, j: (0, 0)),
            pl.BlockSpec((1, HID), lambda i, j: (0, 0)),
        ],
        out_specs=pl.BlockSpec((_BLK, HID), lambda i, j: (j * s_blocks + i, 0)),
        out_shape=jax.ShapeDtypeStruct((N_TOK, HID), jnp.float32),
    )(gathered, pos, w, gamma, beta)


def kernel(input_ids, word_embeddings, position_embeddings, proj_weight, ln_gamma, ln_beta):
    ids_flat = input_ids.reshape(N_TOK)
    gathered = _make_sc_gather()(ids_flat, word_embeddings)
    out = _tc_call(
        gathered,
        position_embeddings,
        proj_weight,
        ln_gamma.reshape(1, HID),
        ln_beta.reshape(1, HID),
    )
    return out.reshape(B, S, HID)


# speedup vs baseline: 2.1035x; 1.0925x over previous
import functools

import jax
import jax.numpy as jnp
from jax import lax
from jax.experimental import pallas as pl
from jax.experimental.pallas import tpu as pltpu
from jax.experimental.pallas import tpu_sc as plsc

VOCAB = 128100
EMB = 512
HID = 1024
B = 4
S = 2048
EPS = 1e-07

N_TOK = B * S

_CHUNK = 128


def _make_sc_gather():
    info = plsc.get_sparse_core_info()
    nc, ns = info.num_cores, info.num_subcores
    nw = nc * ns
    per_w = N_TOK // nw
    n_chunks = per_w // _CHUNK
    mesh = plsc.VectorSubcoreMesh(core_axis_name="c", subcore_axis_name="s")

    @functools.partial(
        pl.kernel,
        mesh=mesh,
        out_type=jax.ShapeDtypeStruct((N_TOK, EMB), jnp.float32),
        scratch_types=[
            pltpu.VMEM((_CHUNK,), jnp.int32),
            pltpu.VMEM((_CHUNK, EMB), jnp.float32),
            pltpu.SemaphoreType.DMA,
        ],
    )
    def gather_k(idx_hbm, table_hbm, out_hbm, idx_v, rows_v, sem):
        wid = lax.axis_index("s") * nc + lax.axis_index("c")
        base0 = wid * per_w
        for c in range(n_chunks):
            base = base0 + c * _CHUNK
            pltpu.sync_copy(idx_hbm.at[pl.ds(base, _CHUNK)], idx_v)
            pltpu.async_copy(table_hbm.at[idx_v], rows_v, sem).wait()
            pltpu.sync_copy(rows_v, out_hbm.at[pl.ds(base, _CHUNK)])

    return gather_k


_BLK = 1024


def _tc_body(g_ref, p_ref, w_ref, gamma_ref, beta_ref, o_ref):
    x = (g_ref[...] + p_ref[...]).astype(jnp.bfloat16)
    y = lax.dot_general(x, w_ref[...].astype(jnp.bfloat16),
                        (((1,), (1,)), ((), ())),
                        preferred_element_type=jnp.float32)
    mean = jnp.mean(y, axis=-1, keepdims=True)
    yc = y - mean
    var = jnp.mean(yc * yc, axis=-1, keepdims=True)
    o_ref[...] = yc * lax.rsqrt(var + EPS) * gamma_ref[...] + beta_ref[...]


def _tc_call(gathered, pos, w, gamma, beta):
    s_blocks = S // _BLK
    return pl.pallas_call(
        _tc_body,
        grid=(s_blocks, B),
        in_specs=[
            pl.BlockSpec((_BLK, EMB), lambda i, j: (j * s_blocks + i, 0)),
            pl.BlockSpec((_BLK, EMB), lambda i, j: (i, 0)),
            pl.BlockSpec((HID, EMB), lambda i, j: (0, 0)),
            pl.BlockSpec((1, HID), lambda i, j: (0, 0)),
            pl.BlockSpec((1, HID), lambda i, j: (0, 0)),
        ],
        out_specs=pl.BlockSpec((_BLK, HID), lambda i, j: (j * s_blocks + i, 0)),
        out_shape=jax.ShapeDtypeStruct((N_TOK, HID), jnp.float32),
    )(gathered, pos, w, gamma, beta)


def kernel(input_ids, word_embeddings, position_embeddings, proj_weight, ln_gamma, ln_beta):
    ids_flat = input_ids.reshape(N_TOK)
    gathered = _make_sc_gather()(ids_flat, word_embeddings)
    out = _tc_call(
        gathered,
        position_embeddings,
        proj_weight,
        ln_gamma.reshape(1, HID),
        ln_beta.reshape(1, HID),
    )
    return out.reshape(B, S, HID)


# speedup vs baseline: 2.1232x; 1.0094x over previous
import functools

import jax
import jax.numpy as jnp
from jax import lax
from jax.experimental import pallas as pl
from jax.experimental.pallas import tpu as pltpu
from jax.experimental.pallas import tpu_sc as plsc

VOCAB = 128100
EMB = 512
HID = 1024
B = 4
S = 2048
EPS = 1e-07

N_TOK = B * S

_CHUNK = 128


def _make_sc_gather():
    info = plsc.get_sparse_core_info()
    nc, ns = info.num_cores, info.num_subcores
    nw = nc * ns
    per_w = N_TOK // nw
    n_chunks = per_w // _CHUNK
    mesh = plsc.VectorSubcoreMesh(core_axis_name="c", subcore_axis_name="s")

    @functools.partial(
        pl.kernel,
        mesh=mesh,
        out_type=jax.ShapeDtypeStruct((N_TOK, EMB), jnp.float32),
        scratch_types=[
            pltpu.VMEM((_CHUNK,), jnp.int32),
            pltpu.VMEM((_CHUNK, EMB), jnp.float32),
            pltpu.SemaphoreType.DMA,
        ],
    )
    def gather_k(idx_hbm, table_hbm, out_hbm, idx_v, rows_v, sem):
        wid = lax.axis_index("s") * nc + lax.axis_index("c")
        base0 = wid * per_w
        for c in range(n_chunks):
            base = base0 + c * _CHUNK
            pltpu.sync_copy(idx_hbm.at[pl.ds(base, _CHUNK)], idx_v)
            pltpu.async_copy(table_hbm.at[idx_v], rows_v, sem).wait()
            pltpu.sync_copy(rows_v, out_hbm.at[pl.ds(base, _CHUNK)])

    return gather_k


_BLK = 2048


def _tc_body(g_ref, p_ref, w_ref, gamma_ref, beta_ref, o_ref):
    x = (g_ref[...] + p_ref[...]).astype(jnp.bfloat16)
    y = lax.dot_general(x, w_ref[...].astype(jnp.bfloat16),
                        (((1,), (1,)), ((), ())),
                        preferred_element_type=jnp.float32)
    mean = jnp.mean(y, axis=-1, keepdims=True)
    yc = y - mean
    var = jnp.mean(yc * yc, axis=-1, keepdims=True)
    o_ref[...] = yc * lax.rsqrt(var + EPS) * gamma_ref[...] + beta_ref[...]


def _tc_call(gathered, pos, w, gamma, beta):
    s_blocks = S // _BLK
    return pl.pallas_call(
        _tc_body,
        grid=(s_blocks, B),
        in_specs=[
            pl.BlockSpec((_BLK, EMB), lambda i, j: (j * s_blocks + i, 0)),
            pl.BlockSpec((_BLK, EMB), lambda i, j: (i, 0)),
            pl.BlockSpec((HID, EMB), lambda i, j: (0, 0)),
            pl.BlockSpec((1, HID), lambda i, j: (0, 0)),
            pl.BlockSpec((1, HID), lambda i, j: (0, 0)),
        ],
        out_specs=pl.BlockSpec((_BLK, HID), lambda i, j: (j * s_blocks + i, 0)),
        out_shape=jax.ShapeDtypeStruct((N_TOK, HID), jnp.float32),
    )(gathered, pos, w, gamma, beta)


def kernel(input_ids, word_embeddings, position_embeddings, proj_weight, ln_gamma, ln_beta):
    ids_flat = input_ids.reshape(N_TOK)
    gathered = _make_sc_gather()(ids_flat, word_embeddings)
    out = _tc_call(
        gathered,
        position_embeddings,
        proj_weight,
        ln_gamma.reshape(1, HID),
        ln_beta.reshape(1, HID),
    )
    return out.reshape(B, S, HID)
